# baseline (speedup 1.0000x reference)
import jax
import jax.numpy as jnp
from jax.experimental import pallas as pl
from jax.experimental.pallas import tpu as pltpu

_N, _C, _L = 64, 81, 8732
_SCALE_XY = 10.0
_SCALE_WH = 5.0
_CH = 1280
_NCH = 7
_WLAST = _L - (_NCH - 1) * _CH
_BIG = 1 << 30


def _row_con(p, g):
    w = p.shape[1]
    m = jnp.full((1, w), -jnp.inf, jnp.float32)
    s = jnp.zeros((1, w), jnp.float32)
    pk = jnp.zeros((1, w), jnp.float32)
    for k in range(0, _C, 8):
        rows = min(8, _C - k)
        slab = p[k:k + rows]
        sm = jnp.max(slab, axis=0, keepdims=True)
        mn = jnp.maximum(m, sm)
        s = s * jnp.exp(m - mn) + jnp.sum(jnp.exp(slab - mn), axis=0,
                                          keepdims=True)
        cidx = k + jax.lax.broadcasted_iota(jnp.int32, slab.shape, 0)
        pk += jnp.sum(jnp.where(cidx == g, slab, 0.0), axis=0, keepdims=True)
        m = mn
    return jnp.maximum(jnp.log(s) + (m - pk), 0.0)


def _octal(lo, hi, trips, count_ge_needed):
    def body(_, carry):
        lo, hi = carry
        gap = hi - lo
        base, rem = gap // 8, gap % 8
        ms, oks = [], []
        for j in range(1, 8):
            mj = lo + base * j + (rem * j) // 8
            ms.append(mj)
            oks.append(count_ge_needed(mj))
        newlo, newhi = lo, hi
        for j in range(7):
            newlo = jnp.where(oks[j], ms[j], newlo)
        for j in reversed(range(7)):
            newhi = jnp.where(oks[j], newhi, ms[j])
        return newlo, newhi

    return jax.lax.fori_loop(0, trips, body, (lo, hi))


def _kernel(plabel_ref, ploc_ref, gloc_ref, glabel_ref, dboxes_ref, out_ref):
    n = pl.program_id(0)
    vb_chunks, cp_chunks, pm_chunks, s1_chunks = [], [], [], []
    for i in range(_NCH):
        w = _CH if i < _NCH - 1 else _WLAST
        sl = slice(i * _CH, i * _CH + w)
        p = plabel_ref[0][:, sl]
        g = glabel_ref[0][:, sl]
        con = _row_con(p, g)

        mask = g > 0
        maskf = mask.astype(jnp.float32)
        conneg = jnp.where(mask, 0.0, con)
        conpos = con * maskf
        vb = jax.lax.bitcast_convert_type(conneg, jnp.int32) & 0x7FFFFFFF

        ploc = ploc_ref[0][:, sl]
        gl = gloc_ref[0][:, sl]
        db = dboxes_ref[0][:, sl]
        gxy = _SCALE_XY * (gl[:2] - db[:2]) / db[2:]
        gwh = _SCALE_WH * jnp.log(gl[2:] / db[2:])
        d = ploc - jnp.concatenate([gxy, gwh], axis=0)
        ad = jnp.abs(d)
        sl1row = jnp.sum(jnp.where(ad < 1.0, 0.5 * d * d, ad - 0.5),
                         axis=0, keepdims=True)

        if w != _CH:
            zf = jnp.zeros((1, _CH - w), jnp.float32)
            zi = jnp.zeros((1, _CH - w), jnp.int32)
            vb = jnp.concatenate([vb, zi], axis=1)
            conpos = jnp.concatenate([conpos, zf], axis=1)
            maskf = jnp.concatenate([maskf, zf], axis=1)
            sl1row = jnp.concatenate([sl1row, zf], axis=1)
        vb_chunks.append(vb)
        cp_chunks.append(conpos)
        pm_chunks.append(maskf)
        s1_chunks.append(jnp.where(pm_chunks[-1] > 0, sl1row, 0.0))

    VB = jnp.concatenate(vb_chunks, axis=0)
    CP = jnp.concatenate(cp_chunks, axis=0)
    PM = jnp.concatenate(pm_chunks, axis=0)
    S1 = jnp.concatenate(s1_chunks, axis=0)

    pos = jnp.sum(PM)
    sl1 = jnp.sum(S1)
    cms = jnp.sum(CP)
    ki = jnp.minimum(3.0 * pos, float(_L)).astype(jnp.int32)

    t, _ = _octal(jnp.int32(0), jnp.int32(0x7F800001), 11,
                  lambda m: jnp.sum((VB >= m).astype(jnp.int32)) >= ki)

    gt = VB > t
    ngt = jnp.sum(gt.astype(jnp.int32))
    vf = jax.lax.bitcast_convert_type(VB, jnp.float32)
    s_gt = jnp.sum(jnp.where(gt, vf, 0.0))
    need = ki - ngt

    t_is0 = t == 0
    need0 = jnp.where(t_is0, need, 0)
    idx = (_CH * jax.lax.broadcasted_iota(jnp.int32, VB.shape, 0)
           + jax.lax.broadcasted_iota(jnp.int32, VB.shape, 1))
    idxz = jnp.where(VB == 0, idx, _BIG)
    lo2, hi2 = _octal(
        jnp.int32(-1), jnp.int32(_L - 1), 5,
        lambda m: jnp.sum((idxz <= m).astype(jnp.int32)) < need0)
    cut = jnp.where(need0 > 0, hi2, -1)
    tie0 = jnp.sum(jnp.where(idxz <= cut, CP, 0.0))
    tval = jax.lax.bitcast_convert_type(t, jnp.float32)
    tval = jnp.where(t >= 0x7F800000, 0.0, tval)
    tie = jnp.where(t_is0, tie0, need.astype(jnp.float32) * tval)

    total = sl1 + cms + s_gt + tie
    rowval = jnp.where(pos > 0,
                       total / jnp.maximum(pos, 1e-6), 0.0) / _N

    @pl.when(n == 0)
    def _():
        out_ref[...] = jnp.zeros_like(out_ref)

    out_ref[...] += rowval.reshape(1, 1)


@jax.jit
def kernel(ploc, plabel, gloc, glabel, dboxes):
    ploc = ploc.astype(jnp.float32)
    plabel = plabel.astype(jnp.float32)
    gloc = gloc.astype(jnp.float32)
    dboxes = dboxes.astype(jnp.float32)
    glabel3 = glabel.astype(jnp.int32).reshape(_N, 1, _L)

    out = pl.pallas_call(
        _kernel,
        grid=(_N,),
        in_specs=[
            pl.BlockSpec((1, _C, _L), lambda n: (n, 0, 0)),
            pl.BlockSpec((1, 4, _L), lambda n: (n, 0, 0)),
            pl.BlockSpec((1, 4, _L), lambda n: (n, 0, 0)),
            pl.BlockSpec((1, 1, _L), lambda n: (n, 0, 0)),
            pl.BlockSpec((1, 4, _L), lambda n: (0, 0, 0)),
        ],
        out_specs=pl.BlockSpec((1, 1), lambda n: (0, 0)),
        out_shape=jax.ShapeDtypeStruct((1, 1), jnp.float32),
        compiler_params=pltpu.CompilerParams(
            dimension_semantics=("arbitrary",)),
    )(plabel, ploc, gloc, glabel3, dboxes)
    return out.reshape(())


# speedup vs baseline: 1.7131x; 1.7131x over previous
import jax
import jax.numpy as jnp
from jax.experimental import pallas as pl
from jax.experimental.pallas import tpu as pltpu

_N, _C, _L = 64, 81, 8732
_SCALE_XY = 10.0
_SCALE_WH = 5.0
_CH = 1280
_NCH = 7
_WLAST = _L - (_NCH - 1) * _CH
_LP = 8832
_WPLAST = _LP - (_NCH - 1) * _CH


def _stage_a(plabel_ref, ploc_ref, gloc_ref, glabel_ref, dboxes_ref,
             vb_ref, cp_ref, acc_ref):
    posc = jnp.float32(0.0)
    sl1s = jnp.float32(0.0)
    cms = jnp.float32(0.0)
    for i in range(_NCH):
        w = _CH if i < _NCH - 1 else _WLAST
        sl = slice(i * _CH, i * _CH + w)
        p = plabel_ref[0][:, sl]
        g = glabel_ref[0][:, sl]
        m = jnp.max(p, axis=0, keepdims=True)
        s = jnp.sum(jnp.exp(p - m), axis=0, keepdims=True)
        cidx = jax.lax.broadcasted_iota(jnp.int32, p.shape, 0)
        picked = jnp.sum(jnp.where(cidx == g, p, 0.0), axis=0, keepdims=True)
        con = jnp.maximum(jnp.log(s) + (m - picked), 0.0)

        mask = g > 0
        conneg = jnp.where(mask, 0.0, con)
        conpos = jnp.where(mask, con, 0.0)
        vb = jax.lax.bitcast_convert_type(conneg, jnp.int32) & 0x7FFFFFFF

        ploc = ploc_ref[0][:, sl]
        gl = gloc_ref[0][:, sl]
        db = dboxes_ref[0][:, sl]
        gxy = _SCALE_XY * (gl[:2] - db[:2]) / db[2:]
        gwh = _SCALE_WH * jnp.log(gl[2:] / db[2:])
        d = ploc - jnp.concatenate([gxy, gwh], axis=0)
        ad = jnp.abs(d)
        sl1row = jnp.sum(jnp.where(ad < 1.0, 0.5 * d * d, ad - 0.5),
                         axis=0, keepdims=True)

        posc += jnp.sum(mask.astype(jnp.float32))
        sl1s += jnp.sum(jnp.where(mask, sl1row, 0.0))
        cms += jnp.sum(conpos)

        if i == _NCH - 1:
            zi = jnp.zeros((1, _WPLAST - _WLAST), jnp.int32)
            zf = jnp.zeros((1, _WPLAST - _WLAST), jnp.float32)
            vb = jnp.concatenate([vb, zi], axis=1)
            conpos = jnp.concatenate([conpos, zf], axis=1)
            so = slice(i * _CH, i * _CH + _WPLAST)
        else:
            so = sl
        vb_ref[0, :, so] = vb
        cp_ref[0, :, so] = conpos

    lane = jax.lax.broadcasted_iota(jnp.int32, (1, 128), 1)
    acc_ref[0] = jnp.where(lane == 0, posc,
                           jnp.where(lane == 1, sl1s,
                                     jnp.where(lane == 2, cms, 0.0)))


def _both_halves(x):
    return x + jnp.concatenate([x[:, 64:], x[:, :64]], axis=1)


def _fold(x4):
    p = jnp.sum(x4, axis=0)
    p = jnp.sum(p, axis=0)
    return jnp.sum(p, axis=0, keepdims=True)


def _stage_b(vb_ref, cp_ref, pos_ref, sl1_ref, cm_ref, out_ref):
    vb = vb_ref[...].reshape(69, 8, 8, 128)
    cp = cp_ref[...].reshape(69, 8, 8, 128)
    pos = pos_ref[...]
    sl1 = sl1_ref[...]
    conmask = cm_ref[...]
    ki = jnp.minimum(3.0 * pos, float(_L)).astype(jnp.int32)

    def cnt(pred4):
        return _both_halves(_fold(pred4.astype(jnp.int32)))

    lo = jnp.zeros_like(ki)
    hi = jnp.full_like(ki, 0x7F800001)

    def body(_, carry):
        lo, hi = carry
        mid = lo + (hi - lo) // 2
        ok = cnt(vb >= mid.reshape(1, 1, 1, 128)) >= ki
        return jnp.where(ok, mid, lo), jnp.where(ok, hi, mid)

    lo, hi = jax.lax.fori_loop(0, 31, body, (lo, hi))
    t = lo

    t4 = t.reshape(1, 1, 1, 128)
    gt = vb > t4
    ngt = cnt(gt)
    vf = jax.lax.bitcast_convert_type(vb, jnp.float32)
    s_gt = _both_halves(_fold(jnp.where(gt, vf, 0.0)))
    need = ki - ngt

    t_is0 = t == 0
    eq0 = vb == 0
    need0 = jnp.where(t_is0, need, 0)
    sh = (69, 8, 8, 128)
    row = (64 * jax.lax.broadcasted_iota(jnp.int32, sh, 0)
           + 8 * jax.lax.broadcasted_iota(jnp.int32, sh, 1)
           + jax.lax.broadcasted_iota(jnp.int32, sh, 2))
    half = (jax.lax.broadcasted_iota(jnp.int32, sh, 3) >= 64).astype(jnp.int32)
    idx = 2 * row + half
    idxz = jnp.where(eq0, idx, 1 << 30)
    lo2 = jnp.full_like(ki, -1)
    hi2 = jnp.full_like(ki, _L - 1)

    def body2(_, carry):
        lo2, hi2 = carry
        mid = lo2 + (hi2 - lo2) // 2
        ok = cnt(idxz <= mid.reshape(1, 1, 1, 128)) >= need0
        return jnp.where(ok, lo2, mid), jnp.where(ok, mid, hi2)

    lo2, hi2 = jax.lax.fori_loop(0, 14, body2, (lo2, hi2))
    cut = jnp.where(need0 > 0, hi2, -1)
    tie0 = _both_halves(_fold(
        jnp.where(eq0 & (idx <= cut.reshape(1, 1, 1, 128)), cp, 0.0)))
    tval = jax.lax.bitcast_convert_type(t, jnp.float32)
    tval = jnp.where(t >= 0x7F800000, 0.0, tval)
    tie = jnp.where(t_is0, tie0, need.astype(jnp.float32) * tval)

    closs = conmask + s_gt + tie
    total = sl1 + closs
    num_mask = (pos > 0).astype(jnp.float32)
    posf = jnp.maximum(pos, 1e-6)
    out_ref[...] = (jnp.sum(total * num_mask / posf) / (2 * _N)).reshape(1, 1)


def _to_packed_t(x):
    return x.reshape(_N, _LP // 2, 2).transpose(1, 2, 0).reshape(_LP // 2,
                                                                 2 * _N)


@jax.jit
def kernel(ploc, plabel, gloc, glabel, dboxes):
    ploc = ploc.astype(jnp.float32)
    plabel = plabel.astype(jnp.float32)
    gloc = gloc.astype(jnp.float32)
    dboxes = dboxes.astype(jnp.float32)
    glabel3 = glabel.astype(jnp.int32).reshape(_N, 1, _L)

    vb, cp, acc = pl.pallas_call(
        _stage_a,
        grid=(_N,),
        in_specs=[
            pl.BlockSpec((1, _C, _L), lambda n: (n, 0, 0)),
            pl.BlockSpec((1, 4, _L), lambda n: (n, 0, 0)),
            pl.BlockSpec((1, 4, _L), lambda n: (n, 0, 0)),
            pl.BlockSpec((1, 1, _L), lambda n: (n, 0, 0)),
            pl.BlockSpec((1, 4, _L), lambda n: (0, 0, 0)),
        ],
        out_specs=[
            pl.BlockSpec((1, 1, _LP), lambda n: (n, 0, 0)),
            pl.BlockSpec((1, 1, _LP), lambda n: (n, 0, 0)),
            pl.BlockSpec((1, 1, 128), lambda n: (n, 0, 0)),
        ],
        out_shape=[
            jax.ShapeDtypeStruct((_N, 1, _LP), jnp.int32),
            jax.ShapeDtypeStruct((_N, 1, _LP), jnp.float32),
            jax.ShapeDtypeStruct((_N, 1, 128), jnp.float32),
        ],
        compiler_params=pltpu.CompilerParams(
            dimension_semantics=("parallel",)),
    )(plabel, ploc, gloc, glabel3, dboxes)

    pos128 = jnp.tile(acc[:, 0, 0], 2).reshape(1, 128)
    sl1128 = jnp.tile(acc[:, 0, 1], 2).reshape(1, 128)
    cm128 = jnp.tile(acc[:, 0, 2], 2).reshape(1, 128)
    out = pl.pallas_call(
        _stage_b,
        out_shape=jax.ShapeDtypeStruct((1, 1), jnp.float32),
    )(_to_packed_t(vb), _to_packed_t(cp), pos128, sl1128, cm128)
    return out.reshape(())
